# Initial kernel scaffold; baseline (speedup 1.0000x reference)
import jax
import jax.numpy as jnp
from jax.experimental import pallas as pl


def kernel(x, table):
    raise NotImplementedError("write your pallas kernel here")


# speedup vs baseline: 10.3716x; 10.3716x over previous
import functools

import jax
import jax.numpy as jnp
from jax import lax
from jax.experimental import pallas as pl
from jax.experimental.pallas import tpu as pltpu
from jax.experimental.pallas import tpu_sc as plsc

V_DIM = 100000
EMB_DIM = 128
BATCH = 16384
CTX = 50

NC = 2
NS = 16
NW = NC * NS
LANES = 16

ROWS_PER_W = BATCH // NW
ROWS_PER_STEP = 2
IDX_PER_STEP = ROWS_PER_STEP * CTX
STEPS = ROWS_PER_W // ROWS_PER_STEP
NJ = EMB_DIM // LANES


def _cbow_body(x_hbm, table_hbm, out_hbm, idx_all, rows_v, out_v, sem0, sem1):
    sems = (sem0, sem1)
    wid = lax.axis_index("s") * NC + lax.axis_index("c")

    pltpu.sync_copy(x_hbm.at[wid], idx_all)

    def gather(step, buf):
        return pltpu.async_copy(
            table_hbm.at[idx_all.at[step]], rows_v.at[buf], sems[buf])

    gather(0, 0)

    def outer(g2, carry):
        for b in range(2):
            g = 2 * g2 + b
            pltpu.make_async_copy(
                table_hbm.at[idx_all.at[g]], rows_v.at[b], sems[b]).wait()

            @pl.when(g < STEPS - 1)
            def _():
                gather(g + 1, 1 - b)

            for r in range(ROWS_PER_STEP):
                base = CTX * r
                accs = tuple(
                    rows_v[b, base, pl.ds(LANES * j, LANES)] for j in range(NJ))

                def inner(c, accs):
                    return tuple(
                        accs[j] + rows_v[b, base + c, pl.ds(LANES * j, LANES)]
                        for j in range(NJ))

                accs = lax.fori_loop(1, CTX, inner, accs)
                for j in range(NJ):
                    out_v[g * ROWS_PER_STEP + r, pl.ds(LANES * j, LANES)] = (
                        accs[j] * (1.0 / CTX))
        return carry

    lax.fori_loop(0, STEPS // 2, outer, 0)

    pltpu.sync_copy(out_v, out_hbm.at[pl.ds(wid * ROWS_PER_W, ROWS_PER_W)])


@jax.jit
def kernel(x, table):
    x3 = x.astype(jnp.int32).reshape(NW, STEPS, IDX_PER_STEP)
    mesh = plsc.VectorSubcoreMesh(core_axis_name="c", subcore_axis_name="s",
                                  num_cores=NC, num_subcores=NS)
    f = pl.kernel(
        _cbow_body,
        out_type=jax.ShapeDtypeStruct((BATCH, EMB_DIM), jnp.float32),
        mesh=mesh,
        scratch_types=[
            pltpu.VMEM((STEPS, IDX_PER_STEP), jnp.int32),
            pltpu.VMEM((2, IDX_PER_STEP, EMB_DIM), jnp.float32),
            pltpu.VMEM((ROWS_PER_W, EMB_DIM), jnp.float32),
            pltpu.SemaphoreType.DMA,
            pltpu.SemaphoreType.DMA,
        ],
    )
    return f(x3, table)
